# Initial kernel scaffold; baseline (speedup 1.0000x reference)
import jax
import jax.numpy as jnp
from jax.experimental import pallas as pl


def kernel(x, gating_output, w1, w2):
    raise NotImplementedError("write your pallas kernel here")


# speedup vs baseline: 1.0133x; 1.0133x over previous
import jax
import jax.numpy as jnp
from jax.experimental import pallas as pl
from jax.experimental.pallas import tpu as pltpu

E = 8
TOPK = 2
D = 1024
DFF = 2048
T = 2048

BT = 128
NA = T * TOPK
NB = NA // BT + E
NP = NB * BT
BC = 128


def _routing_metadata(gating_output):
    scores = jax.nn.softmax(gating_output.astype(jnp.float32), axis=-1)
    topk_w, topk_ids = jax.lax.top_k(scores, TOPK)
    topk_w = topk_w / jnp.sum(topk_w, axis=-1, keepdims=True)

    flat_e = topk_ids.reshape(-1)
    onehot = (flat_e[:, None] == jnp.arange(E)[None, :]).astype(jnp.int32)
    counts = jnp.sum(onehot, axis=0)
    blocks_e = (counts + BT - 1) // BT
    padded_off = (jnp.concatenate([jnp.zeros((1,), jnp.int32),
                                   jnp.cumsum(blocks_e)[:-1].astype(jnp.int32)])
                  * BT)
    rank = jnp.cumsum(onehot, axis=0) - onehot
    rank = jnp.take_along_axis(rank, flat_e[:, None], axis=1)[:, 0]
    pos = padded_off[flat_e] + rank
    pos_t = pos.reshape(T, TOPK).T.reshape(1, TOPK, T)
    block_expert = jnp.repeat(jnp.arange(E, dtype=jnp.int32), blocks_e,
                              total_repeat_length=NB)
    return block_expert, pos_t, topk_w


def _expert_kernel(be_ref, pos_ref, x_ref, w1_ref, w2_ref, y_ref):
    b = pl.program_id(0)
    row_id = jax.lax.broadcasted_iota(jnp.int32, (BT, T), 0) + b * BT
    sel = ((row_id == pos_ref[0, 0, :][None, :]) |
           (row_id == pos_ref[0, 1, :][None, :])
           ).astype(jnp.float32).astype(jnp.bfloat16)
    xb = jax.lax.dot_general(sel, x_ref[...], (((1,), (0,)), ((), ())),
                             preferred_element_type=jnp.float32
                             ).astype(jnp.bfloat16)
    nt = (((1,), (1,)), ((), ()))
    g = jax.lax.dot_general(xb, w1_ref[0, :DFF, :], nt,
                            preferred_element_type=jnp.float32)
    u = jax.lax.dot_general(xb, w1_ref[0, DFF:, :], nt,
                            preferred_element_type=jnp.float32)
    h = (g * jax.nn.sigmoid(g)) * u
    y_ref[...] = jax.lax.dot_general(h.astype(jnp.bfloat16), w2_ref[0], nt,
                                     preferred_element_type=jnp.float32
                                     ).astype(jnp.bfloat16)


def _combine_kernel(pos_ref, w_ref, y_ref, out_ref):
    c = pl.program_id(0)
    col_id = jax.lax.broadcasted_iota(jnp.int32, (BC, NP), 1)
    w0 = w_ref[:, 0:1]
    w1 = w_ref[:, 1:2]
    sel = (jnp.where(col_id == pos_ref[:, 0:1], w0, 0.0) +
           jnp.where(col_id == pos_ref[:, 1:2], w1, 0.0)).astype(jnp.bfloat16)
    out_ref[...] = jax.lax.dot_general(sel, y_ref[...], (((1,), (0,)), ((), ())),
                                       preferred_element_type=jnp.float32)


@jax.jit
def kernel(x, gating_output, w1, w2):
    block_expert, pos_t, topk_w = _routing_metadata(gating_output)
    pos_tok = pos_t.reshape(TOPK, T).T

    xb16 = x.astype(jnp.bfloat16)
    w1b16 = w1.astype(jnp.bfloat16)
    w2b16 = w2.astype(jnp.bfloat16)

    grid_spec = pltpu.PrefetchScalarGridSpec(
        num_scalar_prefetch=1,
        grid=(NB,),
        in_specs=[
            pl.BlockSpec((1, TOPK, T), lambda b, be: (0, 0, 0)),
            pl.BlockSpec((T, D), lambda b, be: (0, 0)),
            pl.BlockSpec((1, 2 * DFF, D), lambda b, be: (be[b], 0, 0)),
            pl.BlockSpec((1, D, DFF), lambda b, be: (be[b], 0, 0)),
        ],
        out_specs=pl.BlockSpec((BT, D), lambda b, be: (b, 0)),
    )
    y_sorted = pl.pallas_call(
        _expert_kernel,
        grid_spec=grid_spec,
        out_shape=jax.ShapeDtypeStruct((NP, D), jnp.bfloat16),
        compiler_params=pltpu.CompilerParams(
            dimension_semantics=("arbitrary",),
        ),
    )(block_expert, pos_t, xb16, w1b16, w2b16)

    out = pl.pallas_call(
        _combine_kernel,
        grid=(T // BC,),
        in_specs=[
            pl.BlockSpec((BC, TOPK), lambda c: (c, 0)),
            pl.BlockSpec((BC, TOPK), lambda c: (c, 0)),
            pl.BlockSpec((NP, D), lambda c: (0, 0)),
        ],
        out_specs=pl.BlockSpec((BC, D), lambda c: (c, 0)),
        out_shape=jax.ShapeDtypeStruct((T, D), jnp.float32),
        compiler_params=pltpu.CompilerParams(
            dimension_semantics=("arbitrary",),
        ),
    )(pos_tok, topk_w, y_sorted)
    return out


# speedup vs baseline: 1.4412x; 1.4222x over previous
import jax
import jax.numpy as jnp
from jax.experimental import pallas as pl
from jax.experimental.pallas import tpu as pltpu

E = 8
TOPK = 2
D = 1024
DFF = 2048
T = 2048

BT = 128
NA = T * TOPK
NB = NA // BT + E
NP = NB * BT
EMAXB = T // BT
DH = DFF // 2
BC = 128


def _routing_kernel(g_ref, pos_ref, w_ref, nblk_ref, base_ref):
    s = g_ref[...]
    lane = jax.lax.broadcasted_iota(jnp.int32, (T, E), 1)
    m = jnp.max(s, axis=1, keepdims=True)
    p = jnp.exp(s - m)
    m1 = jnp.max(p, axis=1, keepdims=True)
    is1 = jnp.min(jnp.where(p == m1, lane, E), axis=1, keepdims=True)
    pm = jnp.where(lane == is1, -1.0, p)
    m2 = jnp.max(pm, axis=1, keepdims=True)
    is2 = jnp.min(jnp.where(pm == m2, lane, E), axis=1, keepdims=True)
    wsum = m1 + m2
    w_ref[:, 0:1] = m1 / wsum
    w_ref[:, 1:2] = m2 / wsum

    onehot = ((lane == is1) | (lane == is2)).astype(jnp.int32)
    c = onehot
    sft = 1
    while sft < T:
        z = jnp.zeros((sft, E), jnp.int32)
        c = c + jnp.concatenate([z, c[:T - sft, :]], axis=0)
        sft *= 2
    counts = c[T - 1:T, :]
    rank = c - onehot
    blocks_e = (counts + BT - 1) // BT
    b = blocks_e
    sft = 1
    while sft < E:
        z = jnp.zeros((1, sft), jnp.int32)
        b = b + jnp.concatenate([z, b[:, :E - sft]], axis=1)
        sft *= 2
    base_excl = b - blocks_e
    nblk_ref[...] = blocks_e
    base_ref[...] = base_excl

    def pick(isel):
        r = jnp.sum(jnp.where(lane == isel, rank, 0), axis=1, keepdims=True)
        bb = jnp.sum(jnp.where(lane == isel, base_excl, 0), axis=1,
                     keepdims=True)
        return bb * BT + r
    pos_ref[:, 0:1] = pick(is1)
    pos_ref[:, 1:2] = pick(is2)


def _expert_kernel(nblk_ref, base_ref, pos_ref, x_ref, w1g_ref, w1u_ref,
                   w2_ref, y_ref, xs_ref):
    e = pl.program_id(0)
    f = pl.program_id(1)
    nb = nblk_ref[e]
    base = base_ref[e]

    @pl.when((e == 0) & (f == 0))
    def _():
        y_ref[...] = jnp.zeros_like(y_ref)

    wg = w1g_ref[0].astype(jnp.bfloat16)
    wu = w1u_ref[0].astype(jnp.bfloat16)
    w2c = w2_ref[0].astype(jnp.bfloat16)

    @pl.when(f == 0)
    def _():
        def gbody(i, _):
            row0 = (base + i) * BT
            row_id = jax.lax.broadcasted_iota(jnp.int32, (BT, T), 0) + row0
            sel = ((row_id == pos_ref[0, 0, :][None, :]) |
                   (row_id == pos_ref[0, 1, :][None, :])
                   ).astype(jnp.float32).astype(jnp.bfloat16)
            xs_ref[i] = jax.lax.dot_general(
                sel, x_ref[...], (((1,), (0,)), ((), ())),
                preferred_element_type=jnp.float32).astype(jnp.bfloat16)
            return 0
        jax.lax.fori_loop(0, nb, gbody, 0)

    nt = (((1,), (1,)), ((), ()))

    def cbody(i, _):
        xb = xs_ref[i]
        g = jax.lax.dot_general(xb, wg, nt, preferred_element_type=jnp.float32)
        u = jax.lax.dot_general(xb, wu, nt, preferred_element_type=jnp.float32)
        h = ((g * jax.nn.sigmoid(g)) * u).astype(jnp.bfloat16)
        yp = jax.lax.dot_general(h, w2c, nt, preferred_element_type=jnp.float32)
        blk = base + i
        prev = y_ref[blk].astype(jnp.float32)
        y_ref[blk] = jnp.where(f == 0, yp, prev + yp).astype(jnp.bfloat16)
        return 0
    jax.lax.fori_loop(0, nb, cbody, 0)


def _combine_kernel(pos_ref, w_ref, y_ref, out_ref):
    col_id = jax.lax.broadcasted_iota(jnp.int32, (BC, NP), 1)
    sel = (jnp.where(col_id == pos_ref[:, 0:1], w_ref[:, 0:1], 0.0) +
           jnp.where(col_id == pos_ref[:, 1:2], w_ref[:, 1:2], 0.0)
           ).astype(jnp.bfloat16)
    out_ref[...] = jax.lax.dot_general(sel, y_ref[...],
                                       (((1,), (0,)), ((), ())),
                                       preferred_element_type=jnp.float32)


@jax.jit
def kernel(x, gating_output, w1, w2):
    pos_tok, topk_w, nblk, base = pl.pallas_call(
        _routing_kernel,
        grid=(1,),
        in_specs=[pl.BlockSpec((T, E), lambda i: (0, 0))],
        out_specs=[
            pl.BlockSpec((T, TOPK), lambda i: (0, 0)),
            pl.BlockSpec((T, TOPK), lambda i: (0, 0)),
            pl.BlockSpec((1, E), lambda i: (0, 0)),
            pl.BlockSpec((1, E), lambda i: (0, 0)),
        ],
        out_shape=[
            jax.ShapeDtypeStruct((T, TOPK), jnp.int32),
            jax.ShapeDtypeStruct((T, TOPK), jnp.float32),
            jax.ShapeDtypeStruct((1, E), jnp.int32),
            jax.ShapeDtypeStruct((1, E), jnp.int32),
        ],
    )(gating_output.astype(jnp.float32))

    pos_t = pos_tok.T.reshape(1, TOPK, T)
    xb16 = x.astype(jnp.bfloat16)

    grid_spec = pltpu.PrefetchScalarGridSpec(
        num_scalar_prefetch=2,
        grid=(E, 2),
        in_specs=[
            pl.BlockSpec((1, TOPK, T), lambda e, f, nb, bs: (0, 0, 0)),
            pl.BlockSpec((T, D), lambda e, f, nb, bs: (0, 0)),
            pl.BlockSpec((1, DH, D), lambda e, f, nb, bs: (e, f, 0)),
            pl.BlockSpec((1, DH, D), lambda e, f, nb, bs: (e, 2 + f, 0)),
            pl.BlockSpec((1, D, DH), lambda e, f, nb, bs: (e, 0, f)),
        ],
        out_specs=pl.BlockSpec((NB, BT, D), lambda e, f, nb, bs: (0, 0, 0)),
        scratch_shapes=[pltpu.VMEM((EMAXB, BT, D), jnp.bfloat16)],
    )
    y_sorted = pl.pallas_call(
        _expert_kernel,
        grid_spec=grid_spec,
        out_shape=jax.ShapeDtypeStruct((NB, BT, D), jnp.bfloat16),
        compiler_params=pltpu.CompilerParams(
            dimension_semantics=("arbitrary", "arbitrary"),
        ),
    )(nblk.reshape(E), base.reshape(E), pos_t, xb16, w1, w1, w2)

    out = pl.pallas_call(
        _combine_kernel,
        grid=(T // BC,),
        in_specs=[
            pl.BlockSpec((BC, TOPK), lambda c: (c, 0)),
            pl.BlockSpec((BC, TOPK), lambda c: (c, 0)),
            pl.BlockSpec((NP, D), lambda c: (0, 0)),
        ],
        out_specs=pl.BlockSpec((BC, D), lambda c: (c, 0)),
        out_shape=jax.ShapeDtypeStruct((T, D), jnp.float32),
        compiler_params=pltpu.CompilerParams(
            dimension_semantics=("arbitrary",),
        ),
    )(pos_tok, topk_w, y_sorted.reshape(NP, D))
    return out
